# Initial kernel scaffold; baseline (speedup 1.0000x reference)
import jax
import jax.numpy as jnp
from jax.experimental import pallas as pl


def kernel(x, wg, We, be):
    raise NotImplementedError("write your pallas kernel here")


# speedup vs baseline: 2.0235x; 2.0235x over previous
import jax
import jax.numpy as jnp
from jax.experimental import pallas as pl
from jax.experimental.pallas import tpu as pltpu

E = 8
D = 1024


def _moe_dense_kernel(x_ref, wg_ref, We_ref, be_ref, out_ref, gate_ref, assign_ref):
    e = pl.program_id(0)

    @pl.when(e == 0)
    def _():
        logits = jnp.dot(x_ref[...], wg_ref[...].T,
                         preferred_element_type=jnp.float32)
        m = jnp.max(logits, axis=1, keepdims=True)
        p = jnp.exp(logits - m)
        gates = p / jnp.sum(p, axis=1, keepdims=True)
        assign = jnp.argmax(gates, axis=1, keepdims=True).astype(jnp.int32)
        gate_ref[...] = jnp.max(gates, axis=1, keepdims=True)
        assign_ref[...] = assign

    sub = jnp.dot(x_ref[...], We_ref[0].T, preferred_element_type=jnp.float32)
    sub = (sub + be_ref[0, 0][None, :]) * gate_ref[...]
    mask = assign_ref[...] == e
    @pl.when(e == 0)
    def _():
        out_ref[...] = jnp.where(mask, sub, jnp.zeros_like(sub))
    @pl.when(e != 0)
    def _():
        out_ref[...] = jnp.where(mask, sub, out_ref[...])


def kernel(x, wg, We, be):
    orig_shape = x.shape
    x2 = x.reshape(-1, x.shape[-1])
    T = x2.shape[0]
    out = pl.pallas_call(
        _moe_dense_kernel,
        grid=(E,),
        in_specs=[
            pl.BlockSpec((T, D), lambda e: (0, 0)),
            pl.BlockSpec((E, D), lambda e: (0, 0)),
            pl.BlockSpec((1, D, D), lambda e: (e, 0, 0)),
            pl.BlockSpec((1, 1, D), lambda e: (e, 0, 0)),
        ],
        out_specs=pl.BlockSpec((T, D), lambda e: (0, 0)),
        out_shape=jax.ShapeDtypeStruct((T, D), jnp.float32),
        scratch_shapes=[
            pltpu.VMEM((T, 1), jnp.float32),
            pltpu.VMEM((T, 1), jnp.int32),
        ],
    )(x2, wg, We, be.reshape(E, 1, D))
    return out.reshape(orig_shape)
